# Initial kernel scaffold; baseline (speedup 1.0000x reference)
import jax
import jax.numpy as jnp
from jax.experimental import pallas as pl


def kernel(input_ids, tok_embed, pos_embed):
    raise NotImplementedError("write your pallas kernel here")


# speedup vs baseline: 1.0598x; 1.0598x over previous
import jax
import jax.numpy as jnp
from jax import lax
from jax.experimental import pallas as pl
from jax.experimental.pallas import tpu as pltpu
from jax.experimental.pallas import tpu_sc as plsc

LANES = 16
W = 32


def kernel(input_ids, tok_embed, pos_embed):
    B, S = input_ids.shape
    H = tok_embed.shape[1]
    n = B * S
    NW = 32
    P = S // NW
    NCH = P // W
    ids = input_ids.reshape(n).astype(jnp.int32)

    mesh = plsc.VectorSubcoreMesh(core_axis_name="c", subcore_axis_name="s")

    @jax.jit
    def run(ids, tok, pos):
        @pl.kernel(
            out_type=jax.ShapeDtypeStruct((n, H), jnp.float32),
            mesh=mesh,
            scratch_types=[
                pltpu.VMEM((B * P,), jnp.int32),
                pltpu.VMEM((W, H), jnp.float32),
                pltpu.VMEM((W, H), jnp.float32),
            ],
        )
        def emb_kernel(ids_hbm, tok_hbm, pos_hbm, out_hbm, idx_v, pos_v,
                       rows_v):
            wid = lax.axis_index("s") * 2 + lax.axis_index("c")
            pbase = wid * P
            for b in range(B):
                pltpu.sync_copy(ids_hbm.at[pl.ds(b * S + pbase, P)],
                                idx_v.at[pl.ds(b * P, P)])

            @pl.loop(0, NCH)
            def _(ch):
                pltpu.sync_copy(pos_hbm.at[pl.ds(pbase + ch * W, W)], pos_v)
                for b in range(B):
                    pltpu.sync_copy(
                        tok_hbm.at[idx_v.at[pl.ds(b * P + ch * W, W)]],
                        rows_v)

                    @pl.loop(0, W)
                    def _(r):
                        for col in range(0, H, LANES):
                            slc = (pl.ds(r, 1), pl.ds(col, LANES))
                            rows_v.at[*slc][...] = (
                                rows_v.at[*slc][...] + pos_v.at[*slc][...]
                            )

                    pltpu.sync_copy(
                        rows_v,
                        out_hbm.at[pl.ds(b * S + pbase + ch * W, W)])

        return emb_kernel(ids, tok, pos)

    out = run(ids, tok_embed, pos_embed)
    return out.reshape(B, S, H)


# speedup vs baseline: 1.4020x; 1.3229x over previous
import jax
import jax.numpy as jnp
from jax import lax
from jax.experimental import pallas as pl
from jax.experimental.pallas import tpu as pltpu
from jax.experimental.pallas import tpu_sc as plsc

LANES = 16
W = 32


def kernel(input_ids, tok_embed, pos_embed):
    B, S = input_ids.shape
    H = tok_embed.shape[1]
    n = B * S
    NW = 32
    P = S // NW
    NCH = P // W
    ids = input_ids.reshape(n).astype(jnp.int32)

    mesh = plsc.VectorSubcoreMesh(core_axis_name="c", subcore_axis_name="s")

    @jax.jit
    def run(ids, tok, pos):
        @pl.kernel(
            out_type=jax.ShapeDtypeStruct((n, H), jnp.float32),
            mesh=mesh,
            scratch_types=[
                pltpu.VMEM((B * P,), jnp.int32),
                pltpu.VMEM((W, H), jnp.float32),
            ]
            + [pltpu.VMEM((W, H), jnp.float32)] * B
            + [pltpu.SemaphoreType.DMA] * (2 * B),
        )
        def emb_kernel(ids_hbm, tok_hbm, pos_hbm, out_hbm, idx_v, pos_v,
                       *bufs_and_sems):
            rows = bufs_and_sems[:B]
            gsem = bufs_and_sems[B:2 * B]
            ssem = bufs_and_sems[2 * B:]
            wid = lax.axis_index("s") * 2 + lax.axis_index("c")
            pbase = wid * P
            for b in range(B):
                pltpu.sync_copy(ids_hbm.at[pl.ds(b * S + pbase, P)],
                                idx_v.at[pl.ds(b * P, P)])

            def gather_start(b, ch):
                pltpu.async_copy(
                    tok_hbm.at[idx_v.at[pl.ds(b * P + ch * W, W)]],
                    rows[b], gsem[b])

            def block_wait(dst, sem):
                pltpu.make_async_copy(pos_hbm.at[pl.ds(0, W)], dst, sem).wait()

            for b in range(B):
                gather_start(b, 0)

            @pl.loop(0, NCH)
            def _(ch):
                pltpu.sync_copy(pos_hbm.at[pl.ds(pbase + ch * W, W)], pos_v)
                for b in range(B):
                    block_wait(rows[b], gsem[b])

                    @pl.loop(0, W)
                    def _(r, _b=b):
                        for col in range(0, H, LANES):
                            slc = (pl.ds(r, 1), pl.ds(col, LANES))
                            plsc.addupdate(rows[_b].at[*slc],
                                           pos_v.at[*slc][...])

                    pltpu.async_copy(
                        rows[b],
                        out_hbm.at[pl.ds(b * S + pbase + ch * W, W)],
                        ssem[b])
                ch_next = lax.min(ch + 1, NCH - 1)
                for b in range(B):
                    block_wait(rows[b], ssem[b])
                    gather_start(b, ch_next)

            for b in range(B):
                block_wait(rows[b], gsem[b])

        return emb_kernel(ids, tok, pos)

    out = run(ids, tok_embed, pos_embed)
    return out.reshape(B, S, H)


# speedup vs baseline: 1.4973x; 1.0680x over previous
import jax
import jax.numpy as jnp
from jax import lax
from jax.experimental import pallas as pl
from jax.experimental.pallas import tpu as pltpu
from jax.experimental.pallas import tpu_sc as plsc

LANES = 16
W = 16
NB = 8


def kernel(input_ids, tok_embed, pos_embed):
    B, S = input_ids.shape
    H = tok_embed.shape[1]
    n = B * S
    NW = 32
    P = S // NW
    NCH = P // W
    ids = input_ids.reshape(n).astype(jnp.int32)

    mesh = plsc.VectorSubcoreMesh(core_axis_name="c", subcore_axis_name="s")

    @jax.jit
    def run(ids, tok, pos):
        @pl.kernel(
            out_type=jax.ShapeDtypeStruct((n, H), jnp.float32),
            mesh=mesh,
            scratch_types=[pltpu.VMEM((B * P,), jnp.int32)]
            + [pltpu.VMEM((W, H), jnp.float32)] * 2
            + [pltpu.VMEM((W, H), jnp.float32)] * NB
            + [pltpu.SemaphoreType.DMA] * (2 + 2 * NB),
        )
        def emb_kernel(ids_hbm, tok_hbm, pos_hbm, out_hbm, idx_v, *rest):
            pos_v = rest[0:2]
            rows = rest[2:2 + NB]
            psem = rest[2 + NB:4 + NB]
            gsem = rest[4 + NB:4 + 2 * NB]
            ssem = rest[4 + 2 * NB:]
            wid = lax.axis_index("s") * 2 + lax.axis_index("c")
            pbase = wid * P
            for b in range(B):
                pltpu.sync_copy(ids_hbm.at[pl.ds(b * S + pbase, P)],
                                idx_v.at[pl.ds(b * P, P)])

            def gather_start(j, b, ch):
                pltpu.async_copy(
                    tok_hbm.at[idx_v.at[pl.ds(b * P + ch * W, W)]],
                    rows[j], gsem[j])

            def pos_start(p, ch):
                pltpu.async_copy(pos_hbm.at[pl.ds(pbase + ch * W, W)],
                                 pos_v[p], psem[p])

            def drain(dst, sem):
                pltpu.make_async_copy(pos_hbm.at[pl.ds(0, W)], dst, sem).wait()

            pos_start(0, 0)
            pos_start(1, 1)
            for j in range(NB):
                gather_start(j, j % B, j // B)

            @pl.loop(0, NCH // 2)
            def _(it):
                c0 = it * 2
                for j in range(NB):
                    p, b = j // B, j % B
                    if j % B == 0:
                        drain(pos_v[p], psem[p])
                    drain(rows[j], gsem[j])

                    @pl.loop(0, W)
                    def _(r, _j=j, _p=p):
                        for col in range(0, H, LANES):
                            slc = (pl.ds(r, 1), pl.ds(col, LANES))
                            plsc.addupdate(rows[_j].at[*slc],
                                           pos_v[_p].at[*slc][...])

                    pltpu.async_copy(
                        rows[j],
                        out_hbm.at[pl.ds(b * S + pbase + (c0 + p) * W, W)],
                        ssem[j])
                    if j % B == B - 1:
                        pos_start(p, lax.min(c0 + p + 2, NCH - 1))
                for j in range(NB):
                    drain(rows[j], ssem[j])
                    gather_start(j, j % B,
                                 lax.min(c0 + 2 + j // B, NCH - 1))

            for p in range(2):
                drain(pos_v[p], psem[p])
            for j in range(NB):
                drain(rows[j], gsem[j])

        return emb_kernel(ids, tok, pos)

    out = run(ids, tok_embed, pos_embed)
    return out.reshape(B, S, H)
